# Initial kernel scaffold; baseline (speedup 1.0000x reference)
import functools
import jax
import jax.numpy as jnp
from jax import lax
from jax.experimental import pallas as pl
from jax.experimental.pallas import tpu as pltpu
from jax.experimental.pallas import tpu_sc as plsc

N = 10000
E = 160000
T = 12
H = 128
NC, NS = 2, 16
NW = NC * NS
EPW = 5120
EPAD = NW * EPW
B = 128
NB = EPW // B
RPT = 625
ZPT = 640
ACC_ROWS = NS * ZPT

_mesh = plsc.VectorSubcoreMesh(core_axis_name="c", subcore_axis_name="s",
                               num_cores=NC, num_subcores=NS)


@functools.partial(
    pl.kernel,
    out_type=jax.ShapeDtypeStruct((NC, N, 16), jnp.float32),
    mesh=_mesh,
    scratch_types=[
        pltpu.VMEM((1, B), jnp.int32),
        pltpu.VMEM((B, 16), jnp.float32),
        pltpu.VMEM((ZPT, 16), jnp.float32),
        pltpu.VMEM_SHARED((ACC_ROWS, 16), jnp.float32),
    ],
)
def _sc_deg(dst_hbm, ones_hbm, z16_hbm, out_hbm, idx_v, ones_v, z_v, acc_sh):
    c = lax.axis_index("c")
    s = lax.axis_index("s")
    w = s * NC + c
    pltpu.sync_copy(ones_hbm, ones_v)
    pltpu.sync_copy(z16_hbm, z_v)
    pltpu.sync_copy(z_v, acc_sh.at[pl.ds(s * ZPT, ZPT)])
    plsc.subcore_barrier()

    def body(i, carry):
        off = w * EPW + i * B
        pltpu.sync_copy(dst_hbm.at[pl.ds(off, B)], idx_v.at[0])
        pltpu.sync_copy(ones_v, acc_sh.at[idx_v.at[0]], add=True)
        return carry

    lax.fori_loop(0, NB, body, 0)
    plsc.subcore_barrier()
    pltpu.sync_copy(acc_sh.at[pl.ds(s * RPT, RPT)],
                    out_hbm.at[c].at[pl.ds(s * RPT, RPT)])


@functools.partial(
    pl.kernel,
    out_type=jax.ShapeDtypeStruct((T, NC, N, H), jnp.float32),
    mesh=_mesh,
    scratch_types=[
        pltpu.VMEM((2, B), jnp.int32),
        pltpu.VMEM((B, H), jnp.float32),
        pltpu.VMEM((128, H), jnp.float32),
        pltpu.VMEM_SHARED((ACC_ROWS, H), jnp.float32),
        pltpu.SemaphoreType.DMA,
    ],
)
def _sc_agg(g_hbm, src_hbm, dst_hbm, zrs_hbm, out_hbm, idx_v, rows_v, z_v,
            acc_sh, sem):
    c = lax.axis_index("c")
    s = lax.axis_index("s")
    w = s * NC + c
    pltpu.sync_copy(zrs_hbm, z_v)
    for t in range(T):
        for k in range(ZPT // 128):
            pltpu.sync_copy(z_v, acc_sh.at[pl.ds(s * ZPT + k * 128, 128)])
        plsc.subcore_barrier()

        def body(i, carry):
            off = w * EPW + i * B
            pltpu.sync_copy(src_hbm.at[pl.ds(off, B)], idx_v.at[0])
            pltpu.sync_copy(dst_hbm.at[pl.ds(off, B)], idx_v.at[1])
            pltpu.async_copy(g_hbm.at[t].at[idx_v.at[0]], rows_v, sem).wait()
            pltpu.sync_copy(rows_v, acc_sh.at[idx_v.at[1]], add=True)
            return carry

        lax.fori_loop(0, NB, body, 0)
        plsc.subcore_barrier()
        pltpu.sync_copy(acc_sh.at[pl.ds(s * RPT, RPT)],
                        out_hbm.at[t, c].at[pl.ds(s * RPT, RPT)])
        plsc.subcore_barrier()


@functools.partial(
    pl.kernel,
    out_type=[jax.ShapeDtypeStruct((EPAD, H), jnp.float32),
              jax.ShapeDtypeStruct((EPAD, H), jnp.float32)],
    mesh=_mesh,
    scratch_types=[
        pltpu.VMEM((2, B), jnp.int32),
        pltpu.VMEM((B, H), jnp.float32),
        pltpu.VMEM((B, H), jnp.float32),
        pltpu.SemaphoreType.DMA,
        pltpu.SemaphoreType.DMA,
    ],
)
def _sc_gather2(h_hbm, u_hbm, v_hbm, hu_hbm, hv_hbm, idx_v, ru_v, rv_v,
                sem_u, sem_v):
    c = lax.axis_index("c")
    s = lax.axis_index("s")
    w = s * NC + c

    def body(i, carry):
        off = w * EPW + i * B
        pltpu.sync_copy(u_hbm.at[pl.ds(off, B)], idx_v.at[0])
        pltpu.sync_copy(v_hbm.at[pl.ds(off, B)], idx_v.at[1])
        cu = pltpu.async_copy(h_hbm.at[idx_v.at[0]], ru_v, sem_u)
        cv = pltpu.async_copy(h_hbm.at[idx_v.at[1]], rv_v, sem_v)
        cu.wait()
        cv.wait()
        pltpu.sync_copy(ru_v, hu_hbm.at[pl.ds(off, B)])
        pltpu.sync_copy(rv_v, hv_hbm.at[pl.ds(off, B)])
        return carry

    lax.fori_loop(0, NB, body, 0)


_R = 1000


def _dinv_body(d_ref, o_ref):
    deg = d_ref[0, :, :1] + d_ref[1, :, :1] + 1.0
    o_ref[...] = jnp.broadcast_to(lax.rsqrt(deg), o_ref.shape)


def _tc_dinv(deg2):
    return pl.pallas_call(
        _dinv_body,
        grid=(N // _R,),
        in_specs=[pl.BlockSpec((NC, _R, 16), lambda n: (0, n, 0))],
        out_specs=pl.BlockSpec((_R, H), lambda n: (n, 0)),
        out_shape=jax.ShapeDtypeStruct((N, H), jnp.float32),
    )(deg2)


def _scale_body(x_ref, d_ref, o_ref):
    o_ref[...] = x_ref[...] * d_ref[...][None]


def _tc_scale(xT, dinv_b):
    return pl.pallas_call(
        _scale_body,
        grid=(T, N // _R),
        in_specs=[pl.BlockSpec((1, _R, H), lambda t, n: (t, n, 0)),
                  pl.BlockSpec((_R, H), lambda t, n: (n, 0))],
        out_specs=pl.BlockSpec((1, _R, H), lambda t, n: (t, n, 0)),
        out_shape=jax.ShapeDtypeStruct((T, N, H), jnp.float32),
    )(xT, dinv_b)


def _layer1_body(a_ref, g_ref, d_ref, w_ref, b_ref, o_ref):
    d = d_ref[...]
    u = (a_ref[0, 0] + a_ref[0, 1] + g_ref[0]) * d
    h = jnp.dot(u, w_ref[...], preferred_element_type=jnp.float32) + b_ref[...]
    o_ref[...] = (jnp.maximum(h, 0.0) * d)[None]


def _tc_layer1(A1, g1, dinv_b, W1, b1r):
    return pl.pallas_call(
        _layer1_body,
        grid=(T, N // _R),
        in_specs=[pl.BlockSpec((1, NC, _R, H), lambda t, n: (t, 0, n, 0)),
                  pl.BlockSpec((1, _R, H), lambda t, n: (t, n, 0)),
                  pl.BlockSpec((_R, H), lambda t, n: (n, 0)),
                  pl.BlockSpec((H, H), lambda t, n: (0, 0)),
                  pl.BlockSpec((1, H), lambda t, n: (0, 0))],
        out_specs=pl.BlockSpec((1, _R, H), lambda t, n: (t, n, 0)),
        out_shape=jax.ShapeDtypeStruct((T, N, H), jnp.float32),
    )(A1, g1, dinv_b, W1, b1r)


def _gru_body(a_ref, g_ref, d_ref, w2_ref, b2_ref, wih_ref, whh_ref,
              bih_ref, bhh_ref, o_ref):
    d = d_ref[...]
    vs = (a_ref[:, 0] + a_ref[:, 1] + g_ref[...]) * d[None]
    cT = (((1,), (1,)), ((), ()))
    w2ih = lax.dot_general(w2_ref[...], wih_ref[...], cT,
                           preferred_element_type=jnp.float32)
    bfold = lax.dot_general(b2_ref[...], wih_ref[...], cT,
                            preferred_element_type=jnp.float32) + bih_ref[...]
    h = jnp.zeros(o_ref.shape, jnp.float32)
    for t in range(T):
        gi = jnp.dot(vs[t], w2ih, preferred_element_type=jnp.float32) + bfold
        gh = lax.dot_general(h, whh_ref[...], cT,
                             preferred_element_type=jnp.float32) + bhh_ref[...]
        r = jax.nn.sigmoid(gi[:, :H] + gh[:, :H])
        z = jax.nn.sigmoid(gi[:, H:2 * H] + gh[:, H:2 * H])
        n = jnp.tanh(gi[:, 2 * H:] + r * gh[:, 2 * H:])
        h = (1.0 - z) * n + z * h
    o_ref[...] = h


def _tc_gru(A2, g2, dinv_b, W2, b2r, Wih, Whh, bihr, bhhr):
    return pl.pallas_call(
        _gru_body,
        grid=(N // _R,),
        in_specs=[pl.BlockSpec((T, NC, _R, H), lambda n: (0, 0, n, 0)),
                  pl.BlockSpec((T, _R, H), lambda n: (0, n, 0)),
                  pl.BlockSpec((_R, H), lambda n: (n, 0)),
                  pl.BlockSpec((H, H), lambda n: (0, 0)),
                  pl.BlockSpec((1, H), lambda n: (0, 0)),
                  pl.BlockSpec((3 * H, H), lambda n: (0, 0)),
                  pl.BlockSpec((3 * H, H), lambda n: (0, 0)),
                  pl.BlockSpec((1, 3 * H), lambda n: (0, 0)),
                  pl.BlockSpec((1, 3 * H), lambda n: (0, 0))],
        out_specs=pl.BlockSpec((_R, H), lambda n: (n, 0)),
        out_shape=jax.ShapeDtypeStruct((N, H), jnp.float32),
    )(A2, g2, dinv_b, W2, b2r, Wih, Whh, bihr, bhhr)


_RE = 2048


def _heads_body(hu_ref, hv_ref, sa_ref, sb_ref, sb1_ref, s2_ref, sb2_ref,
                aa_ref, ab_ref, ab1_ref, a2_ref, ab2_ref, ps_ref, pa_ref):
    hu = hu_ref[...]
    hv = hv_ref[...]
    t1 = jnp.dot(hu, sa_ref[...], preferred_element_type=jnp.float32)
    t1 = t1 + jnp.dot(hv, sb_ref[...], preferred_element_type=jnp.float32)
    t1 = jnp.maximum(t1 + sb1_ref[...], 0.0)
    ps_ref[...] = jnp.sum(t1 * s2_ref[...], axis=1, keepdims=True) + sb2_ref[...]
    t2 = jnp.dot(hu, aa_ref[...], preferred_element_type=jnp.float32)
    t2 = t2 + jnp.dot(hv, ab_ref[...], preferred_element_type=jnp.float32)
    t2 = jnp.maximum(t2 + ab1_ref[...], 0.0)
    pa_ref[...] = jnp.sum(t2 * a2_ref[...], axis=1, keepdims=True) + ab2_ref[...]


def _tc_heads(hu, hv, sW1a, sW1b, sb1r, sW2r, sb2r, aW1a, aW1b, ab1r, aW2r,
              ab2r):
    wspec = pl.BlockSpec((H, H), lambda e: (0, 0))
    bspec = pl.BlockSpec((1, H), lambda e: (0, 0))
    sspec = pl.BlockSpec((1, 1), lambda e: (0, 0))
    return pl.pallas_call(
        _heads_body,
        grid=(EPAD // _RE,),
        in_specs=[pl.BlockSpec((_RE, H), lambda e: (e, 0)),
                  pl.BlockSpec((_RE, H), lambda e: (e, 0)),
                  wspec, wspec, bspec, bspec, sspec,
                  wspec, wspec, bspec, bspec, sspec],
        out_specs=[pl.BlockSpec((_RE, 1), lambda e: (e, 0)),
                   pl.BlockSpec((_RE, 1), lambda e: (e, 0))],
        out_shape=[jax.ShapeDtypeStruct((EPAD, 1), jnp.float32),
                   jax.ShapeDtypeStruct((EPAD, 1), jnp.float32)],
    )(hu, hv, sW1a, sW1b, sb1r, sW2r, sb2r, aW1a, aW1b, ab1r, aW2r, ab2r)


def _pad_edges(a, fill):
    per = E // NW
    return jnp.pad(a.reshape(NW, per), ((0, 0), (0, EPW - per)),
                   constant_values=fill).reshape(-1)


def kernel(x, gW1, gb1, gW2, gb2, Wih, Whh, bih, bhh, sW1, sb1, sW2, sb2,
           aW1, ab1, aW2, ab2, edge_index, edge_mapping_u, edge_mapping_v):
    src = edge_index[0]
    dst = edge_index[1]
    srcp = _pad_edges(src, 0)
    dstp = _pad_edges(dst, N)
    up = _pad_edges(edge_mapping_u, 0)
    vp = _pad_edges(edge_mapping_v, 0)
    zrs = jnp.zeros((128, H), jnp.float32)
    z16 = jnp.zeros((ZPT, 16), jnp.float32)
    ones16 = jnp.ones((B, 16), jnp.float32)

    deg2 = _sc_deg(dstp, ones16, z16)
    dinv_b = _tc_dinv(deg2)
    xT = jnp.swapaxes(x, 0, 1)
    g1 = _tc_scale(xT, dinv_b)
    A1 = _sc_agg(g1, srcp, dstp, zrs)
    g2 = _tc_layer1(A1, g1, dinv_b, gW1, gb1.reshape(1, H))
    A2 = _sc_agg(g2, srcp, dstp, zrs)
    h_n = _tc_gru(A2, g2, dinv_b, gW2, gb2.reshape(1, H), Wih, Whh,
                  bih.reshape(1, 3 * H), bhh.reshape(1, 3 * H))
    hu, hv = _sc_gather2(h_n, up, vp)
    psp, pap = _tc_heads(hu, hv, sW1[:H], sW1[H:], sb1.reshape(1, H),
                         sW2.reshape(1, H), sb2.reshape(1, 1),
                         aW1[:H], aW1[H:], ab1.reshape(1, H),
                         aW2.reshape(1, H), ab2.reshape(1, 1))
    return (psp[:E], pap[:E])


# speedup vs baseline: 6.0705x; 6.0705x over previous
import functools
import jax
import jax.numpy as jnp
from jax import lax
from jax.experimental import pallas as pl
from jax.experimental.pallas import tpu as pltpu
from jax.experimental.pallas import tpu_sc as plsc

N = 10000
E = 160000
T = 12
H = 128
NC, NS = 2, 16
NW = NC * NS
EPW = 5120
EPAD = NW * EPW
B = 128
NB = EPW // B
ZPT = 640
NPAD = NS * ZPT

_mesh = plsc.VectorSubcoreMesh(core_axis_name="c", subcore_axis_name="s",
                               num_cores=NC, num_subcores=NS)


@functools.partial(
    pl.kernel,
    out_type=jax.ShapeDtypeStruct((NC, NPAD, H), jnp.float32),
    mesh=_mesh,
    scratch_types=[
        pltpu.VMEM((1, B), jnp.int32),
        pltpu.VMEM((B, H), jnp.float32),
        pltpu.VMEM((128, H), jnp.float32),
        pltpu.VMEM_SHARED((NPAD, H), jnp.float32),
    ],
)
def _sc_deg(dst_hbm, ones_hbm, zrs_hbm, out_hbm, idx_v, ones_v, z_v, acc_sh):
    c = lax.axis_index("c")
    s = lax.axis_index("s")
    w = s * NC + c
    pltpu.sync_copy(ones_hbm, ones_v)
    pltpu.sync_copy(zrs_hbm, z_v)
    for k in range(ZPT // 128):
        pltpu.sync_copy(z_v, acc_sh.at[pl.ds(s * ZPT + k * 128, 128)])
    plsc.subcore_barrier()

    def body(i, carry):
        off = w * EPW + i * B
        pltpu.sync_copy(dst_hbm.at[pl.ds(off, B)], idx_v.at[0])
        pltpu.sync_copy(ones_v, acc_sh.at[idx_v.at[0]], add=True)
        return carry

    lax.fori_loop(0, NB, body, 0)
    plsc.subcore_barrier()
    pltpu.sync_copy(acc_sh.at[pl.ds(s * ZPT, ZPT)],
                    out_hbm.at[c].at[pl.ds(s * ZPT, ZPT)])


@functools.partial(
    pl.kernel,
    out_type=jax.ShapeDtypeStruct((T * NC * NPAD, H), jnp.float32),
    mesh=_mesh,
    scratch_types=[
        pltpu.VMEM((2, B), jnp.int32),
        pltpu.VMEM((B, H), jnp.float32),
        pltpu.VMEM((128, H), jnp.float32),
        pltpu.VMEM_SHARED((NPAD, H), jnp.float32),
        pltpu.SemaphoreType.DMA,
    ],
)
def _sc_agg(g_hbm, src_hbm, dst_hbm, zrs_hbm, out_hbm, idx_v, rows_v, z_v,
            acc_sh, sem):
    c = lax.axis_index("c")
    s = lax.axis_index("s")
    w = s * NC + c
    pltpu.sync_copy(zrs_hbm, z_v)
    for t in range(T):
        for k in range(ZPT // 128):
            pltpu.sync_copy(z_v, acc_sh.at[pl.ds(s * ZPT + k * 128, 128)])
        plsc.subcore_barrier()

        def body(i, carry):
            off = w * EPW + i * B
            pltpu.sync_copy(src_hbm.at[pl.ds(t * EPAD + off, B)], idx_v.at[0])
            pltpu.sync_copy(dst_hbm.at[pl.ds(off, B)], idx_v.at[1])
            pltpu.async_copy(g_hbm.at[idx_v.at[0]], rows_v, sem).wait()
            pltpu.sync_copy(rows_v, acc_sh.at[idx_v.at[1]], add=True)
            return carry

        lax.fori_loop(0, NB, body, 0)
        plsc.subcore_barrier()
        pltpu.sync_copy(acc_sh.at[pl.ds(s * ZPT, ZPT)],
                        out_hbm.at[pl.ds((t * NC + c) * NPAD + s * ZPT, ZPT)])
        plsc.subcore_barrier()


@functools.partial(
    pl.kernel,
    out_type=[jax.ShapeDtypeStruct((EPAD, H), jnp.float32),
              jax.ShapeDtypeStruct((EPAD, H), jnp.float32)],
    mesh=_mesh,
    scratch_types=[
        pltpu.VMEM((2, B), jnp.int32),
        pltpu.VMEM((B, H), jnp.float32),
        pltpu.VMEM((B, H), jnp.float32),
        pltpu.SemaphoreType.DMA,
        pltpu.SemaphoreType.DMA,
    ],
)
def _sc_gather2(h_hbm, u_hbm, v_hbm, hu_hbm, hv_hbm, idx_v, ru_v, rv_v,
                sem_u, sem_v):
    c = lax.axis_index("c")
    s = lax.axis_index("s")
    w = s * NC + c

    def body(i, carry):
        off = w * EPW + i * B
        pltpu.sync_copy(u_hbm.at[pl.ds(off, B)], idx_v.at[0])
        pltpu.sync_copy(v_hbm.at[pl.ds(off, B)], idx_v.at[1])
        cu = pltpu.async_copy(h_hbm.at[idx_v.at[0]], ru_v, sem_u)
        cv = pltpu.async_copy(h_hbm.at[idx_v.at[1]], rv_v, sem_v)
        cu.wait()
        cv.wait()
        pltpu.sync_copy(ru_v, hu_hbm.at[pl.ds(off, B)])
        pltpu.sync_copy(rv_v, hv_hbm.at[pl.ds(off, B)])
        return carry

    lax.fori_loop(0, NB, body, 0)


_R = 1000


def _dinv_body(d_ref, o_ref):
    deg = d_ref[0, :, :1] + d_ref[1, :, :1] + 1.0
    o_ref[...] = jnp.broadcast_to(1.0 / jnp.sqrt(deg), o_ref.shape)


def _tc_dinv(deg2):
    return pl.pallas_call(
        _dinv_body,
        grid=(N // _R,),
        in_specs=[pl.BlockSpec((NC, _R, H), lambda n: (0, n, 0))],
        out_specs=pl.BlockSpec((_R, H), lambda n: (n, 0)),
        out_shape=jax.ShapeDtypeStruct((N, H), jnp.float32),
    )(deg2)


def _scale_body(x_ref, d_ref, o_ref):
    o_ref[...] = x_ref[...] * d_ref[...][None]


def _tc_scale(xT, dinv_b):
    return pl.pallas_call(
        _scale_body,
        grid=(T, N // _R),
        in_specs=[pl.BlockSpec((1, _R, H), lambda t, n: (t, n, 0)),
                  pl.BlockSpec((_R, H), lambda t, n: (n, 0))],
        out_specs=pl.BlockSpec((1, _R, H), lambda t, n: (t, n, 0)),
        out_shape=jax.ShapeDtypeStruct((T, N, H), jnp.float32),
    )(xT, dinv_b)


def _layer1_body(a_ref, g_ref, d_ref, w_ref, b_ref, o_ref):
    d = d_ref[...]
    u = (a_ref[0, 0] + a_ref[0, 1] + g_ref[0]) * d
    h = jnp.dot(u, w_ref[...], preferred_element_type=jnp.float32, precision=lax.Precision.HIGHEST) + b_ref[...]
    o_ref[...] = (jnp.maximum(h, 0.0) * d)[None]


def _tc_layer1(A1, g1, dinv_b, W1, b1r):
    return pl.pallas_call(
        _layer1_body,
        grid=(T, N // _R),
        in_specs=[pl.BlockSpec((1, NC, _R, H), lambda t, n: (t, 0, n, 0)),
                  pl.BlockSpec((1, _R, H), lambda t, n: (t, n, 0)),
                  pl.BlockSpec((_R, H), lambda t, n: (n, 0)),
                  pl.BlockSpec((H, H), lambda t, n: (0, 0)),
                  pl.BlockSpec((1, H), lambda t, n: (0, 0))],
        out_specs=pl.BlockSpec((1, _R, H), lambda t, n: (t, n, 0)),
        out_shape=jax.ShapeDtypeStruct((T, N, H), jnp.float32),
    )(A1, g1, dinv_b, W1, b1r)


def _gru_body(a_ref, g_ref, d_ref, w2_ref, b2_ref, wih_ref, whh_ref,
              bih_ref, bhh_ref, o_ref):
    d = d_ref[...]
    vs = (a_ref[:, 0] + a_ref[:, 1] + g_ref[...]) * d[None]
    cT = (((1,), (1,)), ((), ()))
    w2ih = lax.dot_general(w2_ref[...], wih_ref[...], cT,
                           preferred_element_type=jnp.float32, precision=lax.Precision.HIGHEST)
    bfold = lax.dot_general(b2_ref[...], wih_ref[...], cT,
                            preferred_element_type=jnp.float32, precision=lax.Precision.HIGHEST) + bih_ref[...]
    h = jnp.zeros(o_ref.shape, jnp.float32)
    for t in range(T):
        gi = jnp.dot(vs[t], w2ih, preferred_element_type=jnp.float32, precision=lax.Precision.HIGHEST) + bfold
        gh = lax.dot_general(h, whh_ref[...], cT,
                             preferred_element_type=jnp.float32, precision=lax.Precision.HIGHEST) + bhh_ref[...]
        r = jax.nn.sigmoid(gi[:, :H] + gh[:, :H])
        z = jax.nn.sigmoid(gi[:, H:2 * H] + gh[:, H:2 * H])
        n = jnp.tanh(gi[:, 2 * H:] + r * gh[:, 2 * H:])
        h = (1.0 - z) * n + z * h
    o_ref[...] = h


def _tc_gru(A2, g2, dinv_b, W2, b2r, Wih, Whh, bihr, bhhr):
    return pl.pallas_call(
        _gru_body,
        grid=(N // _R,),
        in_specs=[pl.BlockSpec((T, NC, _R, H), lambda n: (0, 0, n, 0)),
                  pl.BlockSpec((T, _R, H), lambda n: (0, n, 0)),
                  pl.BlockSpec((_R, H), lambda n: (n, 0)),
                  pl.BlockSpec((H, H), lambda n: (0, 0)),
                  pl.BlockSpec((1, H), lambda n: (0, 0)),
                  pl.BlockSpec((3 * H, H), lambda n: (0, 0)),
                  pl.BlockSpec((3 * H, H), lambda n: (0, 0)),
                  pl.BlockSpec((1, 3 * H), lambda n: (0, 0)),
                  pl.BlockSpec((1, 3 * H), lambda n: (0, 0))],
        out_specs=pl.BlockSpec((_R, H), lambda n: (n, 0)),
        out_shape=jax.ShapeDtypeStruct((N, H), jnp.float32),
    )(A2, g2, dinv_b, W2, b2r, Wih, Whh, bihr, bhhr)


_RE = 2048


def _heads_body(hu_ref, hv_ref, sa_ref, sb_ref, sb1_ref, s2_ref, sb2_ref,
                aa_ref, ab_ref, ab1_ref, a2_ref, ab2_ref, ps_ref, pa_ref):
    hu = hu_ref[...]
    hv = hv_ref[...]
    t1 = jnp.dot(hu, sa_ref[...], preferred_element_type=jnp.float32, precision=lax.Precision.HIGHEST)
    t1 = t1 + jnp.dot(hv, sb_ref[...], preferred_element_type=jnp.float32, precision=lax.Precision.HIGHEST)
    t1 = jnp.maximum(t1 + sb1_ref[...], 0.0)
    ps_ref[...] = jnp.sum(t1 * s2_ref[...], axis=1, keepdims=True) + sb2_ref[...]
    t2 = jnp.dot(hu, aa_ref[...], preferred_element_type=jnp.float32, precision=lax.Precision.HIGHEST)
    t2 = t2 + jnp.dot(hv, ab_ref[...], preferred_element_type=jnp.float32, precision=lax.Precision.HIGHEST)
    t2 = jnp.maximum(t2 + ab1_ref[...], 0.0)
    pa_ref[...] = jnp.sum(t2 * a2_ref[...], axis=1, keepdims=True) + ab2_ref[...]


def _tc_heads(hu, hv, sW1a, sW1b, sb1r, sW2r, sb2r, aW1a, aW1b, ab1r, aW2r,
              ab2r):
    wspec = pl.BlockSpec((H, H), lambda e: (0, 0))
    bspec = pl.BlockSpec((1, H), lambda e: (0, 0))
    sspec = pl.BlockSpec((1, 1), lambda e: (0, 0))
    return pl.pallas_call(
        _heads_body,
        grid=(EPAD // _RE,),
        in_specs=[pl.BlockSpec((_RE, H), lambda e: (e, 0)),
                  pl.BlockSpec((_RE, H), lambda e: (e, 0)),
                  wspec, wspec, bspec, bspec, sspec,
                  wspec, wspec, bspec, bspec, sspec],
        out_specs=[pl.BlockSpec((_RE, 1), lambda e: (e, 0)),
                   pl.BlockSpec((_RE, 1), lambda e: (e, 0))],
        out_shape=[jax.ShapeDtypeStruct((EPAD, 1), jnp.float32),
                   jax.ShapeDtypeStruct((EPAD, 1), jnp.float32)],
    )(hu, hv, sW1a, sW1b, sb1r, sW2r, sb2r, aW1a, aW1b, ab1r, aW2r, ab2r)


def _pad_edges(a, fill):
    per = E // NW
    return jnp.pad(a.reshape(NW, per), ((0, 0), (0, EPW - per)),
                   constant_values=fill).reshape(-1)


def kernel(x, gW1, gb1, gW2, gb2, Wih, Whh, bih, bhh, sW1, sb1, sW2, sb2,
           aW1, ab1, aW2, ab2, edge_index, edge_mapping_u, edge_mapping_v):
    src = edge_index[0]
    dst = edge_index[1]
    srcp = _pad_edges(src, 0)
    dstp = _pad_edges(dst, N)
    up = _pad_edges(edge_mapping_u, 0)
    vp = _pad_edges(edge_mapping_v, 0)
    zrs = jnp.zeros((128, H), jnp.float32)
    ones_r = jnp.ones((B, H), jnp.float32)

    deg2 = _sc_deg(dstp, ones_r, zrs)
    dinv_b = _tc_dinv(deg2)
    xT = jnp.swapaxes(x, 0, 1)
    g1 = _tc_scale(xT, dinv_b)

    srcp_t = (srcp[None, :] + (jnp.arange(T, dtype=jnp.int32) * N)[:, None]
              ).reshape(-1)

    def _agg(G):
        flat = _sc_agg(G.reshape(T * N, H), srcp_t, dstp, zrs)
        return flat.reshape(T, NC, NPAD, H)

    A1 = _agg(g1)
    g2 = _tc_layer1(A1, g1, dinv_b, gW1, gb1.reshape(1, H))
    A2 = _agg(g2)
    h_n = _tc_gru(A2, g2, dinv_b, gW2, gb2.reshape(1, H), Wih, Whh,
                  bih.reshape(1, 3 * H), bhh.reshape(1, 3 * H))
    hu, hv = _sc_gather2(h_n, up, vp)
    psp, pap = _tc_heads(hu, hv, sW1[:H], sW1[H:], sb1.reshape(1, H),
                         sW2.reshape(1, H), sb2.reshape(1, 1),
                         aW1[:H], aW1[H:], ab1.reshape(1, H),
                         aW2.reshape(1, H), ab2.reshape(1, 1))
    per = E // NW
    ps = psp.reshape(NW, EPW, 1)[:, :per].reshape(E, 1)
    pa = pap.reshape(NW, EPW, 1)[:, :per].reshape(E, 1)
    return (ps, pa)


# speedup vs baseline: 7.1398x; 1.1762x over previous
import functools
import jax
import jax.numpy as jnp
from jax import lax
from jax.experimental import pallas as pl
from jax.experimental.pallas import tpu as pltpu
from jax.experimental.pallas import tpu_sc as plsc

N = 10000
E = 160000
T = 12
H = 128
NC, NS = 2, 16
NW = NC * NS
EPW = 5120
EPAD = NW * EPW
B = 128
NB = EPW // B
ZPT = 640
NPAD = NS * ZPT

_mesh = plsc.VectorSubcoreMesh(core_axis_name="c", subcore_axis_name="s",
                               num_cores=NC, num_subcores=NS)


@functools.partial(
    pl.kernel,
    out_type=jax.ShapeDtypeStruct((NC, NPAD, H), jnp.float32),
    mesh=_mesh,
    scratch_types=[
        pltpu.VMEM((1, B), jnp.int32),
        pltpu.VMEM((B, H), jnp.float32),
        pltpu.VMEM((64, H), jnp.float32),
        pltpu.VMEM_SHARED((NPAD, H), jnp.float32),
    ],
)
def _sc_deg(dst_hbm, ones_hbm, z16_hbm, out_hbm, idx_v, ones_v, z_v, acc_sh):
    c = lax.axis_index("c")
    s = lax.axis_index("s")
    w = s * NC + c
    pltpu.sync_copy(ones_hbm, ones_v)
    pltpu.sync_copy(z16_hbm, z_v)
    for k in range(ZPT // 64):
        pltpu.sync_copy(z_v, acc_sh.at[pl.ds(s * ZPT + k * 64, 64)])
    plsc.subcore_barrier()

    def body(i, carry):
        off = w * EPW + i * B
        pltpu.sync_copy(dst_hbm.at[pl.ds(off, B)], idx_v.at[0])
        pltpu.sync_copy(ones_v, acc_sh.at[idx_v.at[0]], add=True)
        return carry

    lax.fori_loop(0, NB, body, 0)
    plsc.subcore_barrier()
    pltpu.sync_copy(acc_sh.at[pl.ds(s * ZPT, ZPT)],
                    out_hbm.at[c].at[pl.ds(s * ZPT, ZPT)])


RING = 2
NOUT = NB // RING


@functools.partial(
    pl.kernel,
    out_type=jax.ShapeDtypeStruct((T, NC, NPAD, H), jnp.float32),
    mesh=_mesh,
    scratch_types=[
        pltpu.VMEM((RING, B), jnp.int32),
        pltpu.VMEM((RING, B), jnp.int32),
        pltpu.VMEM((RING, B, H), jnp.float32),
        pltpu.VMEM((64, H), jnp.float32),
        pltpu.VMEM_SHARED((NPAD, H), jnp.float32),
        pltpu.SemaphoreType.DMA,
        pltpu.SemaphoreType.DMA,
    ],
)
def _sc_agg(g_hbm, src_hbm, dst_hbm, zrs_hbm, out_hbm, isrc_v, idst_v,
            rows_v, z_v, acc_sh, sem0, sem1):
    gsem = (sem0, sem1)
    c = lax.axis_index("c")
    s = lax.axis_index("s")
    w = s * NC + c
    pltpu.sync_copy(zrs_hbm, z_v)
    for t in range(T):
        for k in range(ZPT // 64):
            pltpu.sync_copy(z_v, acc_sh.at[pl.ds(s * ZPT + k * 64, 64)])
        plsc.subcore_barrier()

        for r in range(RING):
            off = w * EPW + r * B
            pltpu.sync_copy(src_hbm.at[pl.ds(t * EPAD + off, B)],
                            isrc_v.at[r])
            pltpu.sync_copy(dst_hbm.at[pl.ds(off, B)], idst_v.at[r])
            pltpu.async_copy(g_hbm.at[isrc_v.at[r]], rows_v.at[r], gsem[r])

        def lap(k, carry):
            for r in range(RING):
                pltpu.make_async_copy(g_hbm.at[isrc_v.at[r]], rows_v.at[r],
                                      gsem[r]).wait()
                pltpu.sync_copy(rows_v.at[r], acc_sh.at[idst_v.at[r]],
                                add=True)

                @pl.when(k < NOUT - 1)
                def _():
                    off = w * EPW + (k + 1) * RING * B + r * B
                    pltpu.sync_copy(src_hbm.at[pl.ds(t * EPAD + off, B)],
                                    isrc_v.at[r])
                    pltpu.sync_copy(dst_hbm.at[pl.ds(off, B)], idst_v.at[r])
                    pltpu.async_copy(g_hbm.at[isrc_v.at[r]], rows_v.at[r],
                                     gsem[r])
            return carry

        lax.fori_loop(0, NOUT, lap, 0)
        plsc.subcore_barrier()
        pltpu.sync_copy(acc_sh.at[pl.ds(s * ZPT, ZPT)],
                        out_hbm.at[t, c].at[pl.ds(s * ZPT, ZPT)])
        plsc.subcore_barrier()


@functools.partial(
    pl.kernel,
    out_type=[jax.ShapeDtypeStruct((EPAD, H), jnp.float32),
              jax.ShapeDtypeStruct((EPAD, H), jnp.float32)],
    mesh=_mesh,
    scratch_types=[
        pltpu.VMEM((2, B), jnp.int32),
        pltpu.VMEM((B, H), jnp.float32),
        pltpu.VMEM((B, H), jnp.float32),
        pltpu.SemaphoreType.DMA,
        pltpu.SemaphoreType.DMA,
    ],
)
def _sc_gather2(h_hbm, u_hbm, v_hbm, hu_hbm, hv_hbm, idx_v, ru_v, rv_v,
                sem_u, sem_v):
    c = lax.axis_index("c")
    s = lax.axis_index("s")
    w = s * NC + c

    def body(i, carry):
        off = w * EPW + i * B
        pltpu.sync_copy(u_hbm.at[pl.ds(off, B)], idx_v.at[0])
        pltpu.sync_copy(v_hbm.at[pl.ds(off, B)], idx_v.at[1])
        cu = pltpu.async_copy(h_hbm.at[idx_v.at[0]], ru_v, sem_u)
        cv = pltpu.async_copy(h_hbm.at[idx_v.at[1]], rv_v, sem_v)
        cu.wait()
        cv.wait()
        pltpu.sync_copy(ru_v, hu_hbm.at[pl.ds(off, B)])
        pltpu.sync_copy(rv_v, hv_hbm.at[pl.ds(off, B)])
        return carry

    lax.fori_loop(0, NB, body, 0)


_R = 1000


def _dinv_body(d_ref, o_ref):
    deg = d_ref[0, :, :1] + d_ref[1, :, :1] + 1.0
    o_ref[...] = jnp.broadcast_to(1.0 / jnp.sqrt(deg), o_ref.shape)


def _tc_dinv(deg2):
    return pl.pallas_call(
        _dinv_body,
        grid=(N // _R,),
        in_specs=[pl.BlockSpec((NC, _R, H), lambda n: (0, n, 0))],
        out_specs=pl.BlockSpec((_R, H), lambda n: (n, 0)),
        out_shape=jax.ShapeDtypeStruct((N, H), jnp.float32),
    )(deg2)


def _scale_body(x_ref, d_ref, o_ref):
    o_ref[...] = x_ref[...] * d_ref[...][None]


def _tc_scale(xT, dinv_b):
    return pl.pallas_call(
        _scale_body,
        grid=(T, N // _R),
        in_specs=[pl.BlockSpec((1, _R, H), lambda t, n: (t, n, 0)),
                  pl.BlockSpec((_R, H), lambda t, n: (n, 0))],
        out_specs=pl.BlockSpec((1, _R, H), lambda t, n: (t, n, 0)),
        out_shape=jax.ShapeDtypeStruct((T, N, H), jnp.float32),
    )(xT, dinv_b)


def _layer1_body(a_ref, g_ref, d_ref, w_ref, b_ref, o_ref):
    d = d_ref[...]
    u = (a_ref[0, 0] + a_ref[0, 1] + g_ref[0]) * d
    h = jnp.dot(u, w_ref[...], preferred_element_type=jnp.float32, precision=lax.Precision.HIGHEST) + b_ref[...]
    o_ref[...] = (jnp.maximum(h, 0.0) * d)[None]


def _tc_layer1(A1, g1, dinv_b, W1, b1r):
    return pl.pallas_call(
        _layer1_body,
        grid=(T, N // _R),
        in_specs=[pl.BlockSpec((1, NC, _R, H), lambda t, n: (t, 0, n, 0)),
                  pl.BlockSpec((1, _R, H), lambda t, n: (t, n, 0)),
                  pl.BlockSpec((_R, H), lambda t, n: (n, 0)),
                  pl.BlockSpec((H, H), lambda t, n: (0, 0)),
                  pl.BlockSpec((1, H), lambda t, n: (0, 0))],
        out_specs=pl.BlockSpec((1, _R, H), lambda t, n: (t, n, 0)),
        out_shape=jax.ShapeDtypeStruct((T, N, H), jnp.float32),
    )(A1, g1, dinv_b, W1, b1r)


def _gru_body(a_ref, g_ref, d_ref, w2_ref, b2_ref, wih_ref, whh_ref,
              bih_ref, bhh_ref, o_ref):
    d = d_ref[...]
    vs = (a_ref[:, 0] + a_ref[:, 1] + g_ref[...]) * d[None]
    cT = (((1,), (1,)), ((), ()))
    w2ih = lax.dot_general(w2_ref[...], wih_ref[...], cT,
                           preferred_element_type=jnp.float32, precision=lax.Precision.HIGHEST)
    bfold = lax.dot_general(b2_ref[...], wih_ref[...], cT,
                            preferred_element_type=jnp.float32, precision=lax.Precision.HIGHEST) + bih_ref[...]
    h = jnp.zeros(o_ref.shape, jnp.float32)
    for t in range(T):
        gi = jnp.dot(vs[t], w2ih, preferred_element_type=jnp.float32, precision=lax.Precision.HIGHEST) + bfold
        gh = lax.dot_general(h, whh_ref[...], cT,
                             preferred_element_type=jnp.float32, precision=lax.Precision.HIGHEST) + bhh_ref[...]
        r = jax.nn.sigmoid(gi[:, :H] + gh[:, :H])
        z = jax.nn.sigmoid(gi[:, H:2 * H] + gh[:, H:2 * H])
        n = jnp.tanh(gi[:, 2 * H:] + r * gh[:, 2 * H:])
        h = (1.0 - z) * n + z * h
    o_ref[...] = h


def _tc_gru(A2, g2, dinv_b, W2, b2r, Wih, Whh, bihr, bhhr):
    return pl.pallas_call(
        _gru_body,
        grid=(N // _R,),
        in_specs=[pl.BlockSpec((T, NC, _R, H), lambda n: (0, 0, n, 0)),
                  pl.BlockSpec((T, _R, H), lambda n: (0, n, 0)),
                  pl.BlockSpec((_R, H), lambda n: (n, 0)),
                  pl.BlockSpec((H, H), lambda n: (0, 0)),
                  pl.BlockSpec((1, H), lambda n: (0, 0)),
                  pl.BlockSpec((3 * H, H), lambda n: (0, 0)),
                  pl.BlockSpec((3 * H, H), lambda n: (0, 0)),
                  pl.BlockSpec((1, 3 * H), lambda n: (0, 0)),
                  pl.BlockSpec((1, 3 * H), lambda n: (0, 0))],
        out_specs=pl.BlockSpec((_R, H), lambda n: (n, 0)),
        out_shape=jax.ShapeDtypeStruct((N, H), jnp.float32),
    )(A2, g2, dinv_b, W2, b2r, Wih, Whh, bihr, bhhr)


_RE = 2048


def _heads_body(hu_ref, hv_ref, sa_ref, sb_ref, sb1_ref, s2_ref, sb2_ref,
                aa_ref, ab_ref, ab1_ref, a2_ref, ab2_ref, ps_ref, pa_ref):
    hu = hu_ref[...]
    hv = hv_ref[...]
    t1 = jnp.dot(hu, sa_ref[...], preferred_element_type=jnp.float32, precision=lax.Precision.HIGHEST)
    t1 = t1 + jnp.dot(hv, sb_ref[...], preferred_element_type=jnp.float32, precision=lax.Precision.HIGHEST)
    t1 = jnp.maximum(t1 + sb1_ref[...], 0.0)
    ps_ref[...] = jnp.sum(t1 * s2_ref[...], axis=1, keepdims=True) + sb2_ref[...]
    t2 = jnp.dot(hu, aa_ref[...], preferred_element_type=jnp.float32, precision=lax.Precision.HIGHEST)
    t2 = t2 + jnp.dot(hv, ab_ref[...], preferred_element_type=jnp.float32, precision=lax.Precision.HIGHEST)
    t2 = jnp.maximum(t2 + ab1_ref[...], 0.0)
    pa_ref[...] = jnp.sum(t2 * a2_ref[...], axis=1, keepdims=True) + ab2_ref[...]


def _tc_heads(hu, hv, sW1a, sW1b, sb1r, sW2r, sb2r, aW1a, aW1b, ab1r, aW2r,
              ab2r):
    wspec = pl.BlockSpec((H, H), lambda e: (0, 0))
    bspec = pl.BlockSpec((1, H), lambda e: (0, 0))
    sspec = pl.BlockSpec((1, 1), lambda e: (0, 0))
    return pl.pallas_call(
        _heads_body,
        grid=(EPAD // _RE,),
        in_specs=[pl.BlockSpec((_RE, H), lambda e: (e, 0)),
                  pl.BlockSpec((_RE, H), lambda e: (e, 0)),
                  wspec, wspec, bspec, bspec, sspec,
                  wspec, wspec, bspec, bspec, sspec],
        out_specs=[pl.BlockSpec((_RE, 1), lambda e: (e, 0)),
                   pl.BlockSpec((_RE, 1), lambda e: (e, 0))],
        out_shape=[jax.ShapeDtypeStruct((EPAD, 1), jnp.float32),
                   jax.ShapeDtypeStruct((EPAD, 1), jnp.float32)],
    )(hu, hv, sW1a, sW1b, sb1r, sW2r, sb2r, aW1a, aW1b, ab1r, aW2r, ab2r)


def _pad_edges(a, fill):
    per = E // NW
    return jnp.pad(a.reshape(NW, per), ((0, 0), (0, EPW - per)),
                   constant_values=fill).reshape(-1)


def kernel(x, gW1, gb1, gW2, gb2, Wih, Whh, bih, bhh, sW1, sb1, sW2, sb2,
           aW1, ab1, aW2, ab2, edge_index, edge_mapping_u, edge_mapping_v):
    src = edge_index[0]
    dst = edge_index[1]
    srcp = _pad_edges(src, 0)
    dstp = _pad_edges(dst, N)
    up = _pad_edges(edge_mapping_u, 0)
    vp = _pad_edges(edge_mapping_v, 0)
    zrs = jnp.zeros((64, H), jnp.float32)
    ones_r = jnp.ones((B, H), jnp.float32)
    srcp_t = (srcp[None, :] + (jnp.arange(T, dtype=jnp.int32) * N)[:, None]
              ).reshape(-1)

    deg2 = _sc_deg(dstp, ones_r, zrs)
    dinv_b = _tc_dinv(deg2)
    xT = jnp.swapaxes(x, 0, 1)
    g1 = _tc_scale(xT, dinv_b)

    def _agg(G):
        return _sc_agg(G.reshape(T * N, H), srcp_t, dstp, zrs)

    A1 = _agg(g1)
    g2 = _tc_layer1(A1, g1, dinv_b, gW1, gb1.reshape(1, H))
    A2 = _agg(g2)
    h_n = _tc_gru(A2, g2, dinv_b, gW2, gb2.reshape(1, H), Wih, Whh,
                  bih.reshape(1, 3 * H), bhh.reshape(1, 3 * H))
    hu, hv = _sc_gather2(h_n, up, vp)
    psp, pap = _tc_heads(hu, hv, sW1[:H], sW1[H:], sb1.reshape(1, H),
                         sW2.reshape(1, H), sb2.reshape(1, 1),
                         aW1[:H], aW1[H:], ab1.reshape(1, H),
                         aW2.reshape(1, H), ab2.reshape(1, 1))
    per = E // NW
    ps = psp.reshape(NW, EPW, 1)[:, :per].reshape(E, 1)
    pa = pap.reshape(NW, EPW, 1)[:, :per].reshape(E, 1)
    return (ps, pa)


# speedup vs baseline: 7.2556x; 1.0162x over previous
import functools
import jax
import jax.numpy as jnp
from jax import lax
from jax.experimental import pallas as pl
from jax.experimental.pallas import tpu as pltpu
from jax.experimental.pallas import tpu_sc as plsc

N = 10000
E = 160000
T = 12
H = 128
NC, NS = 2, 16
NW = NC * NS
EPW = 5120
EPAD = NW * EPW
B = 128
NB = EPW // B
ZPT = 640
NPAD = NS * ZPT

_mesh = plsc.VectorSubcoreMesh(core_axis_name="c", subcore_axis_name="s",
                               num_cores=NC, num_subcores=NS)


@functools.partial(
    pl.kernel,
    out_type=jax.ShapeDtypeStruct((NC, NPAD, H), jnp.float32),
    mesh=_mesh,
    scratch_types=[
        pltpu.VMEM((1, B), jnp.int32),
        pltpu.VMEM((B, H), jnp.float32),
        pltpu.VMEM((64, H), jnp.float32),
        pltpu.VMEM_SHARED((NPAD, H), jnp.float32),
    ],
)
def _sc_deg(dst_hbm, ones_hbm, z16_hbm, out_hbm, idx_v, ones_v, z_v, acc_sh):
    c = lax.axis_index("c")
    s = lax.axis_index("s")
    w = s * NC + c
    pltpu.sync_copy(ones_hbm, ones_v)
    pltpu.sync_copy(z16_hbm, z_v)
    for k in range(ZPT // 64):
        pltpu.sync_copy(z_v, acc_sh.at[pl.ds(s * ZPT + k * 64, 64)])
    plsc.subcore_barrier()

    def body(i, carry):
        off = w * EPW + i * B
        pltpu.sync_copy(dst_hbm.at[pl.ds(off, B)], idx_v.at[0])
        pltpu.sync_copy(ones_v, acc_sh.at[idx_v.at[0]], add=True)
        return carry

    lax.fori_loop(0, NB, body, 0)
    plsc.subcore_barrier()
    pltpu.sync_copy(acc_sh.at[pl.ds(s * ZPT, ZPT)],
                    out_hbm.at[c].at[pl.ds(s * ZPT, ZPT)])


RING = 2
NOUT = NB // RING


@functools.partial(
    pl.kernel,
    out_type=jax.ShapeDtypeStruct((T, NC, NPAD, H), jnp.float32),
    mesh=_mesh,
    scratch_types=[
        pltpu.VMEM((RING, 2, B), jnp.int32),
        pltpu.VMEM((RING, B, H), jnp.float32),
        pltpu.VMEM((64, H), jnp.float32),
        pltpu.VMEM_SHARED((NPAD, H), jnp.float32),
    ] + [pltpu.SemaphoreType.DMA] * (2 * RING),
)
def _sc_agg(g_hbm, cmb_hbm, zrs_hbm, out_hbm, cidx_v, rows_v, z_v,
            acc_sh, *sems):
    gsem = sems[:RING]
    ssem = sems[RING:]
    c = lax.axis_index("c")
    s = lax.axis_index("s")
    w = s * NC + c
    pltpu.sync_copy(zrs_hbm, z_v)
    for t in range(T):
        for k in range(ZPT // 64):
            pltpu.sync_copy(z_v, acc_sh.at[pl.ds(s * ZPT + k * 64, 64)])
        plsc.subcore_barrier()

        for r in range(RING):
            row = ((t * NW + w) * NB + r) * 2
            pltpu.sync_copy(cmb_hbm.at[pl.ds(row, 2)], cidx_v.at[r])
            pltpu.async_copy(g_hbm.at[cidx_v.at[r, 0]], rows_v.at[r],
                             gsem[r])

        def lap(k, carry):
            scat = []
            for r in range(RING):
                pltpu.make_async_copy(g_hbm.at[cidx_v.at[r, 0]],
                                      rows_v.at[r], gsem[r]).wait()
                scat.append(pltpu.async_copy(
                    rows_v.at[r], acc_sh.at[cidx_v.at[r, 1]],
                    ssem[r], add=True))
            for r in range(RING):
                scat[r].wait()

                @pl.when(k < NOUT - 1)
                def _():
                    row = ((t * NW + w) * NB + (k + 1) * RING + r) * 2
                    pltpu.sync_copy(cmb_hbm.at[pl.ds(row, 2)], cidx_v.at[r])
                    pltpu.async_copy(g_hbm.at[cidx_v.at[r, 0]],
                                     rows_v.at[r], gsem[r])
            return carry

        lax.fori_loop(0, NOUT, lap, 0)
        plsc.subcore_barrier()
        pltpu.sync_copy(acc_sh.at[pl.ds(s * ZPT, ZPT)],
                        out_hbm.at[t, c].at[pl.ds(s * ZPT, ZPT)])
        plsc.subcore_barrier()


@functools.partial(
    pl.kernel,
    out_type=[jax.ShapeDtypeStruct((EPAD, H), jnp.float32),
              jax.ShapeDtypeStruct((EPAD, H), jnp.float32)],
    mesh=_mesh,
    scratch_types=[
        pltpu.VMEM((2, B), jnp.int32),
        pltpu.VMEM((B, H), jnp.float32),
        pltpu.VMEM((B, H), jnp.float32),
        pltpu.SemaphoreType.DMA,
        pltpu.SemaphoreType.DMA,
    ],
)
def _sc_gather2(h_hbm, u_hbm, v_hbm, hu_hbm, hv_hbm, idx_v, ru_v, rv_v,
                sem_u, sem_v):
    c = lax.axis_index("c")
    s = lax.axis_index("s")
    w = s * NC + c

    def body(i, carry):
        off = w * EPW + i * B
        pltpu.sync_copy(u_hbm.at[pl.ds(off, B)], idx_v.at[0])
        pltpu.sync_copy(v_hbm.at[pl.ds(off, B)], idx_v.at[1])
        cu = pltpu.async_copy(h_hbm.at[idx_v.at[0]], ru_v, sem_u)
        cv = pltpu.async_copy(h_hbm.at[idx_v.at[1]], rv_v, sem_v)
        cu.wait()
        cv.wait()
        pltpu.sync_copy(ru_v, hu_hbm.at[pl.ds(off, B)])
        pltpu.sync_copy(rv_v, hv_hbm.at[pl.ds(off, B)])
        return carry

    lax.fori_loop(0, NB, body, 0)


_R = 1000


def _dinv_body(d_ref, o_ref):
    deg = d_ref[0, :, :1] + d_ref[1, :, :1] + 1.0
    o_ref[...] = jnp.broadcast_to(1.0 / jnp.sqrt(deg), o_ref.shape)


def _tc_dinv(deg2):
    return pl.pallas_call(
        _dinv_body,
        grid=(N // _R,),
        in_specs=[pl.BlockSpec((NC, _R, H), lambda n: (0, n, 0))],
        out_specs=pl.BlockSpec((_R, H), lambda n: (n, 0)),
        out_shape=jax.ShapeDtypeStruct((N, H), jnp.float32),
    )(deg2)


def _scale_body(x_ref, d_ref, o_ref):
    o_ref[...] = x_ref[...] * d_ref[...][None]


def _tc_scale(xT, dinv_b):
    return pl.pallas_call(
        _scale_body,
        grid=(T, N // _R),
        in_specs=[pl.BlockSpec((1, _R, H), lambda t, n: (t, n, 0)),
                  pl.BlockSpec((_R, H), lambda t, n: (n, 0))],
        out_specs=pl.BlockSpec((1, _R, H), lambda t, n: (t, n, 0)),
        out_shape=jax.ShapeDtypeStruct((T, N, H), jnp.float32),
    )(xT, dinv_b)


def _layer1_body(a_ref, g_ref, d_ref, w_ref, b_ref, o_ref):
    d = d_ref[...]
    u = (a_ref[0, 0] + a_ref[0, 1] + g_ref[0]) * d
    h = jnp.dot(u, w_ref[...], preferred_element_type=jnp.float32, precision=lax.Precision.HIGHEST) + b_ref[...]
    o_ref[...] = (jnp.maximum(h, 0.0) * d)[None]


def _tc_layer1(A1, g1, dinv_b, W1, b1r):
    return pl.pallas_call(
        _layer1_body,
        grid=(T, N // _R),
        in_specs=[pl.BlockSpec((1, NC, _R, H), lambda t, n: (t, 0, n, 0)),
                  pl.BlockSpec((1, _R, H), lambda t, n: (t, n, 0)),
                  pl.BlockSpec((_R, H), lambda t, n: (n, 0)),
                  pl.BlockSpec((H, H), lambda t, n: (0, 0)),
                  pl.BlockSpec((1, H), lambda t, n: (0, 0))],
        out_specs=pl.BlockSpec((1, _R, H), lambda t, n: (t, n, 0)),
        out_shape=jax.ShapeDtypeStruct((T, N, H), jnp.float32),
    )(A1, g1, dinv_b, W1, b1r)


def _gru_body(a_ref, g_ref, d_ref, w2_ref, b2_ref, wih_ref, whh_ref,
              bih_ref, bhh_ref, o_ref):
    d = d_ref[...]
    vs = (a_ref[:, 0] + a_ref[:, 1] + g_ref[...]) * d[None]
    cT = (((1,), (1,)), ((), ()))
    w2ih = lax.dot_general(w2_ref[...], wih_ref[...], cT,
                           preferred_element_type=jnp.float32, precision=lax.Precision.HIGHEST)
    bfold = lax.dot_general(b2_ref[...], wih_ref[...], cT,
                            preferred_element_type=jnp.float32, precision=lax.Precision.HIGHEST) + bih_ref[...]
    h = jnp.zeros(o_ref.shape, jnp.float32)
    for t in range(T):
        gi = jnp.dot(vs[t], w2ih, preferred_element_type=jnp.float32, precision=lax.Precision.HIGHEST) + bfold
        gh = lax.dot_general(h, whh_ref[...], cT,
                             preferred_element_type=jnp.float32, precision=lax.Precision.HIGHEST) + bhh_ref[...]
        r = jax.nn.sigmoid(gi[:, :H] + gh[:, :H])
        z = jax.nn.sigmoid(gi[:, H:2 * H] + gh[:, H:2 * H])
        n = jnp.tanh(gi[:, 2 * H:] + r * gh[:, 2 * H:])
        h = (1.0 - z) * n + z * h
    o_ref[...] = h


def _tc_gru(A2, g2, dinv_b, W2, b2r, Wih, Whh, bihr, bhhr):
    return pl.pallas_call(
        _gru_body,
        grid=(N // _R,),
        in_specs=[pl.BlockSpec((T, NC, _R, H), lambda n: (0, 0, n, 0)),
                  pl.BlockSpec((T, _R, H), lambda n: (0, n, 0)),
                  pl.BlockSpec((_R, H), lambda n: (n, 0)),
                  pl.BlockSpec((H, H), lambda n: (0, 0)),
                  pl.BlockSpec((1, H), lambda n: (0, 0)),
                  pl.BlockSpec((3 * H, H), lambda n: (0, 0)),
                  pl.BlockSpec((3 * H, H), lambda n: (0, 0)),
                  pl.BlockSpec((1, 3 * H), lambda n: (0, 0)),
                  pl.BlockSpec((1, 3 * H), lambda n: (0, 0))],
        out_specs=pl.BlockSpec((_R, H), lambda n: (n, 0)),
        out_shape=jax.ShapeDtypeStruct((N, H), jnp.float32),
    )(A2, g2, dinv_b, W2, b2r, Wih, Whh, bihr, bhhr)


_RE = 2048


def _heads_body(hu_ref, hv_ref, sa_ref, sb_ref, sb1_ref, s2_ref, sb2_ref,
                aa_ref, ab_ref, ab1_ref, a2_ref, ab2_ref, ps_ref, pa_ref):
    hu = hu_ref[...]
    hv = hv_ref[...]
    t1 = jnp.dot(hu, sa_ref[...], preferred_element_type=jnp.float32, precision=lax.Precision.HIGHEST)
    t1 = t1 + jnp.dot(hv, sb_ref[...], preferred_element_type=jnp.float32, precision=lax.Precision.HIGHEST)
    t1 = jnp.maximum(t1 + sb1_ref[...], 0.0)
    ps_ref[...] = jnp.sum(t1 * s2_ref[...], axis=1, keepdims=True) + sb2_ref[...]
    t2 = jnp.dot(hu, aa_ref[...], preferred_element_type=jnp.float32, precision=lax.Precision.HIGHEST)
    t2 = t2 + jnp.dot(hv, ab_ref[...], preferred_element_type=jnp.float32, precision=lax.Precision.HIGHEST)
    t2 = jnp.maximum(t2 + ab1_ref[...], 0.0)
    pa_ref[...] = jnp.sum(t2 * a2_ref[...], axis=1, keepdims=True) + ab2_ref[...]


def _tc_heads(hu, hv, sW1a, sW1b, sb1r, sW2r, sb2r, aW1a, aW1b, ab1r, aW2r,
              ab2r):
    wspec = pl.BlockSpec((H, H), lambda e: (0, 0))
    bspec = pl.BlockSpec((1, H), lambda e: (0, 0))
    sspec = pl.BlockSpec((1, 1), lambda e: (0, 0))
    return pl.pallas_call(
        _heads_body,
        grid=(EPAD // _RE,),
        in_specs=[pl.BlockSpec((_RE, H), lambda e: (e, 0)),
                  pl.BlockSpec((_RE, H), lambda e: (e, 0)),
                  wspec, wspec, bspec, bspec, sspec,
                  wspec, wspec, bspec, bspec, sspec],
        out_specs=[pl.BlockSpec((_RE, 1), lambda e: (e, 0)),
                   pl.BlockSpec((_RE, 1), lambda e: (e, 0))],
        out_shape=[jax.ShapeDtypeStruct((EPAD, 1), jnp.float32),
                   jax.ShapeDtypeStruct((EPAD, 1), jnp.float32)],
    )(hu, hv, sW1a, sW1b, sb1r, sW2r, sb2r, aW1a, aW1b, ab1r, aW2r, ab2r)


def _pad_edges(a, fill):
    per = E // NW
    return jnp.pad(a.reshape(NW, per), ((0, 0), (0, EPW - per)),
                   constant_values=fill).reshape(-1)


def kernel(x, gW1, gb1, gW2, gb2, Wih, Whh, bih, bhh, sW1, sb1, sW2, sb2,
           aW1, ab1, aW2, ab2, edge_index, edge_mapping_u, edge_mapping_v):
    src = edge_index[0]
    dst = edge_index[1]
    srcp = _pad_edges(src, 0)
    dstp = _pad_edges(dst, N)
    up = _pad_edges(edge_mapping_u, 0)
    vp = _pad_edges(edge_mapping_v, 0)
    zrs = jnp.zeros((64, H), jnp.float32)
    ones_r = jnp.ones((B, H), jnp.float32)
    srcp_t = (srcp[None, :] + (jnp.arange(T, dtype=jnp.int32) * N)[:, None]
              ).reshape(T, NW, NB, 1, B)
    dstp_b = jnp.broadcast_to(dstp.reshape(1, NW, NB, 1, B),
                              (T, NW, NB, 1, B))
    cmb = jnp.concatenate([srcp_t, dstp_b], axis=3).reshape(T * NW * NB * 2, B)

    deg2 = _sc_deg(dstp, ones_r, zrs)
    dinv_b = _tc_dinv(deg2)
    xT = jnp.swapaxes(x, 0, 1)
    g1 = _tc_scale(xT, dinv_b)

    def _agg(G):
        return _sc_agg(G.reshape(T * N, H), cmb, zrs)

    A1 = _agg(g1)
    g2 = _tc_layer1(A1, g1, dinv_b, gW1, gb1.reshape(1, H))
    A2 = _agg(g2)
    h_n = _tc_gru(A2, g2, dinv_b, gW2, gb2.reshape(1, H), Wih, Whh,
                  bih.reshape(1, 3 * H), bhh.reshape(1, 3 * H))
    hu, hv = _sc_gather2(h_n, up, vp)
    psp, pap = _tc_heads(hu, hv, sW1[:H], sW1[H:], sb1.reshape(1, H),
                         sW2.reshape(1, H), sb2.reshape(1, 1),
                         aW1[:H], aW1[H:], ab1.reshape(1, H),
                         aW2.reshape(1, H), ab2.reshape(1, 1))
    per = E // NW
    ps = psp.reshape(NW, EPW, 1)[:, :per].reshape(E, 1)
    pa = pap.reshape(NW, EPW, 1)[:, :per].reshape(E, 1)
    return (ps, pa)
